# baseline (speedup 1.0000x reference)
import functools

import jax
import jax.numpy as jnp
from jax import lax
from jax.experimental import pallas as pl
from jax.experimental.pallas import tpu as pltpu
from jax.experimental.pallas import tpu_sc as plsc

_D = 64
_BLK = 256
_NC, _NS, _L = 2, 16, 16
_NW = _NC * _NS
_SENT = _D * _D * _D
_T = 262656

_OFF = [(dx, dy, dz) for dx in (-1, 0, 1) for dy in (-1, 0, 1) for dz in (-1, 0, 1)]

_SC_PARAMS = pltpu.CompilerParams(use_tc_tiling_on_sc=False,
                                  needs_layout_passes=False)
_ZROWS = 16384


def _build_lookup(lin_a1, vals, pad_n):
    p1 = pad_n // _NS
    ts = _T // _NS
    mesh = plsc.VectorSubcoreMesh(
        core_axis_name="c", subcore_axis_name="s", num_cores=1)

    @functools.partial(
        pl.kernel,
        out_type=jax.ShapeDtypeStruct((_T,), jnp.int32),
        mesh=mesh,
        compiler_params=_SC_PARAMS,
        scratch_types=[
            pltpu.VMEM((ts,), jnp.int32),
            pltpu.VMEM((p1,), jnp.int32),
            pltpu.VMEM((p1,), jnp.int32),
            pltpu.SemaphoreType.DMA,
        ],
    )
    def k(lin_hbm, vals_hbm, lookup_hbm, fillv, linv, valsv, sem):
        wid = lax.axis_index("s")
        neg1 = jnp.full((_L,), -1, jnp.int32)

        def fill_body(i, c):
            fillv[pl.ds(i * _L, _L)] = neg1
            return c

        lax.fori_loop(0, ts // _L, fill_body, 0)
        pltpu.sync_copy(fillv, lookup_hbm.at[pl.ds(wid * ts, ts)])
        plsc.subcore_barrier()
        pltpu.sync_copy(lin_hbm.at[pl.ds(wid * p1, p1)], linv)
        pltpu.sync_copy(vals_hbm.at[pl.ds(wid * p1, p1)], valsv)
        pltpu.async_copy(valsv, lookup_hbm.at[linv], sem).wait()

    return k(lin_a1, vals)


def _neighbor_idx(lin_a2, lookup, pad_n):
    p = pad_n // _NW
    ng = p // _L
    e = p * 27
    zr = pad_n
    mesh = plsc.VectorSubcoreMesh(core_axis_name="c", subcore_axis_name="s")

    @functools.partial(
        pl.kernel,
        out_type=jax.ShapeDtypeStruct((27 * pad_n,), jnp.int32),
        mesh=mesh,
        compiler_params=_SC_PARAMS,
        scratch_types=[
            pltpu.VMEM((p,), jnp.int32),
            pltpu.VMEM((e,), jnp.int32),
            pltpu.VMEM((e,), jnp.int32),
            [pltpu.SemaphoreType.DMA] * 4,
        ],
    )
    def k(lin_hbm, lookup_hbm, fidx_hbm, linself, linbuf, rawbuf, sem):
        wid = lax.axis_index("s") * _NC + lax.axis_index("c")
        base = wid * p
        pltpu.sync_copy(lin_hbm.at[pl.ds(base, p)], linself)

        def g_body(g, c):
            lin16 = linself[pl.ds(g * _L, _L)]
            x = jnp.right_shift(lin16, 12)
            y = jnp.bitwise_and(jnp.right_shift(lin16, 6), 63)
            z = jnp.bitwise_and(lin16, 63)
            for kk, (dx, dy, dz) in enumerate(_OFF):
                inb = None
                for comp, dd in ((x, dx), (y, dy), (z, dz)):
                    if dd == -1:
                        m = comp >= 1
                    elif dd == 1:
                        m = comp <= _D - 2
                    else:
                        continue
                    inb = m if inb is None else jnp.logical_and(inb, m)
                nlin = lin16 + (dx * 4096 + dy * 64 + dz)
                if inb is not None:
                    nlin = jnp.where(inb, nlin, _SENT)
                linbuf[pl.ds(kk * p + g * _L, _L)] = nlin
            return c

        lax.fori_loop(0, ng, g_body, 0)
        quarter = e // 4
        descs = [
            pltpu.async_copy(
                lookup_hbm.at[linbuf.at[pl.ds(j * quarter, quarter)]],
                rawbuf.at[pl.ds(j * quarter, quarter)], sem[j])
            for j in range(4)
        ]
        for d in descs:
            d.wait()

        iota16 = lax.iota(jnp.int32, _L)

        def f_body(v, c):
            r = rawbuf[pl.ds(v * _L, _L)]
            spread = jnp.bitwise_and(v * _L + iota16, _ZROWS - 1)
            rawbuf[pl.ds(v * _L, _L)] = jnp.where(r >= 0, r, zr + spread)
            return c

        lax.fori_loop(0, e // _L, f_body, 0)
        for kk in range(27):
            pltpu.sync_copy(rawbuf.at[pl.ds(kk * p, p)],
                            fidx_hbm.at[pl.ds(kk * pad_n + base, p)])

    return k(lin_a2, lookup)


_RING = 4


def _gather_scalars_local(x_pad, fidx, pad_n):
    rows_n = fidx.shape[0]
    r = x_pad.shape[0]
    p = pad_n // _NW
    e = p * 27
    eq = e // 2
    mesh = plsc.VectorSubcoreMesh(core_axis_name="c", subcore_axis_name="s")

    @functools.partial(
        pl.kernel,
        out_type=jax.ShapeDtypeStruct((rows_n,), jnp.float32),
        mesh=mesh,
        compiler_params=_SC_PARAMS,
        scratch_types=[
            pltpu.VMEM((r,), jnp.float32),
            pltpu.VMEM((e,), jnp.int32),
            pltpu.VMEM((eq,), jnp.float32),
            pltpu.SemaphoreType.DMA,
        ],
    )
    def k(x_hbm, fidx_hbm, g_hbm, xloc, idxv, outv, sem):
        wid = lax.axis_index("s") * _NC + lax.axis_index("c")
        base = wid * e
        pltpu.sync_copy(x_hbm, xloc)
        pltpu.sync_copy(fidx_hbm.at[pl.ds(base, e)], idxv)

        def quarter(qi, c):
            def body(v, c2):
                i16 = idxv[pl.ds(qi * eq + v * _L, _L)]
                outv[pl.ds(v * _L, _L)] = plsc.load_gather(xloc, [i16])
                return c2

            lax.fori_loop(0, eq // _L, body, 0)
            pltpu.sync_copy(outv, g_hbm.at[pl.ds(base + qi * eq, eq)])
            return c

        lax.fori_loop(0, 2, quarter, 0)

    return k(x_pad, fidx)


def _gather_rows(x_pad, fidx, pad_n):
    rows_n = fidx.shape[0]
    flat = x_pad.ndim == 1
    c = 1 if flat else x_pad.shape[1]
    p = pad_n // _NW
    h = p // 2
    nch = 54
    dt = x_pad.dtype
    out_sds = jax.ShapeDtypeStruct((rows_n,) if flat else (rows_n, c), dt)
    mesh = plsc.VectorSubcoreMesh(core_axis_name="c", subcore_axis_name="s")

    @functools.partial(
        pl.kernel,
        out_type=out_sds,
        mesh=mesh,
        compiler_params=_SC_PARAMS,
        scratch_types=[
            [pltpu.VMEM((h,), jnp.int32)] * _RING,
            [pltpu.VMEM((h,) if flat else (h, c), dt)] * _RING,
            [pltpu.SemaphoreType.DMA] * _RING,
        ],
    )
    def k(x_hbm, fidx_hbm, g_hbm, idxs, rows, sems):
        wid = lax.axis_index("s") * _NC + lax.axis_index("c")
        base = wid * p

        def off(i):
            return (i // 2) * pad_n + base + (i % 2) * h

        descs = [None] * _RING

        def fire(i, b):
            pltpu.sync_copy(fidx_hbm.at[pl.ds(off(i), h)], idxs[b])
            descs[b] = pltpu.async_copy(x_hbm.at[idxs[b]], rows[b], sems[b])

        for r in range(_RING):
            fire(r, r)
        for i in range(nch):
            b = i % _RING
            descs[b].wait()
            pltpu.sync_copy(rows[b], g_hbm.at[pl.ds(off(i), h)])
            if i + _RING < nch:
                fire(i + _RING, b)

    return k(x_pad, fidx)


def _mm1(g1, w1, b1, pad_n):
    g3d = g1.reshape(27, pad_n // 128, 128)
    eye = jnp.eye(128, dtype=jnp.float32)
    r16 = jnp.repeat(eye, 16, axis=1)
    w1t = jnp.tile(w1[:, 0, :], (1, 128))
    b1t = jnp.tile(b1, 128).reshape(1, 2048)
    blk_p = 1024
    rb = blk_p // 128
    nb = pad_n // blk_p
    nz = _ZROWS // blk_p

    def mk(g_ref, r_ref, w_ref, b_ref, o_ref):
        pid = pl.program_id(0)

        @pl.when(pid < nb)
        def _():
            acc = jnp.zeros((rb, 2048), jnp.float32)
            for kk in range(27):
                e = jnp.dot(g_ref[kk], r_ref[...],
                            preferred_element_type=jnp.float32)
                acc += e * w_ref[kk]
            o_ref[...] = acc + b_ref[...]

        @pl.when(pid >= nb)
        def _():
            o_ref[...] = jnp.zeros_like(o_ref)

    out = pl.pallas_call(
        mk,
        grid=(nb + nz,),
        in_specs=[
            pl.BlockSpec((27, rb, 128), lambda i: (0, jnp.minimum(i, nb - 1), 0)),
            pl.BlockSpec((128, 2048), lambda i: (0, 0)),
            pl.BlockSpec((27, 2048), lambda i: (0, 0)),
            pl.BlockSpec((1, 2048), lambda i: (0, 0)),
        ],
        out_specs=pl.BlockSpec((rb, 2048), lambda i: (i, 0)),
        out_shape=jax.ShapeDtypeStruct(((pad_n + _ZROWS) // 128, 2048),
                                       jnp.float32),
    )(g3d, r16, w1t, b1t)
    return out.reshape(pad_n + _ZROWS, 16)


def _mm_mid(g, w, b, pad_n, cin, cout, blk_p, out_dtype=jnp.float32):
    ppr = 128 // cin
    opr = 256 // cout
    g3d = g.reshape(27, pad_n * cin // 128, 128)
    eye = jnp.eye(ppr, dtype=jnp.float32)
    wbig = jnp.einsum("jm,kcd->kjcmd", eye, w).reshape(27, 128, opr * cout)
    bbig = jnp.tile(b, opr).reshape(1, opr * cout)
    rb = blk_p * cin // 128
    orb = blk_p // opr
    nb = pad_n // blk_p
    nz = _ZROWS // blk_p

    def mk(g_ref, w_ref, b_ref, o_ref):
        pid = pl.program_id(0)

        @pl.when(pid < nb)
        def _():
            acc = jnp.zeros((rb, opr * cout), jnp.float32)
            for kk in range(27):
                acc += jnp.dot(g_ref[kk], w_ref[kk],
                               preferred_element_type=jnp.float32)
            o_ref[...] = (acc + b_ref[...]).astype(out_dtype)

        @pl.when(pid >= nb)
        def _():
            o_ref[...] = jnp.zeros_like(o_ref)

    out = pl.pallas_call(
        mk,
        grid=(nb + nz,),
        in_specs=[
            pl.BlockSpec((27, rb, 128), lambda i: (0, jnp.minimum(i, nb - 1), 0)),
            pl.BlockSpec((27, 128, opr * cout), lambda i: (0, 0, 0)),
            pl.BlockSpec((1, opr * cout), lambda i: (0, 0)),
        ],
        out_specs=pl.BlockSpec((orb, opr * cout), lambda i: (i, 0)),
        out_shape=jax.ShapeDtypeStruct(
            ((pad_n + _ZROWS) // opr, opr * cout), out_dtype),
    )(g3d, wbig, bbig)
    return out.reshape(pad_n + _ZROWS, cout)


def _mm3_head(g, w3, b3, wm, bm, pad_n):
    g3d = g.reshape(27, pad_n * 32 // 128, 128)
    eye4 = jnp.eye(4, dtype=jnp.float32)
    w3big = jnp.einsum("jm,kcd->kjcmd", eye4, w3).reshape(27, 128, 256)
    b3big = jnp.tile(b3, 4).reshape(1, 256)
    wmbig = jnp.einsum("jm,cd->jcmd", eye4, wm).reshape(256, 12)
    bmbig = jnp.tile(bm, 4).reshape(1, 12)
    blk_p = 256
    rb = blk_p * 32 // 128
    nb = pad_n // blk_p

    def mk(g_ref, w_ref, b_ref, wm_ref, bm_ref, o_ref):
        acc = jnp.zeros((rb, 256), jnp.float32)
        for kk in range(27):
            acc += jnp.dot(g_ref[kk].astype(jnp.float32), w_ref[kk],
                           preferred_element_type=jnp.float32)
        t = acc + b_ref[...]
        o_ref[...] = jnp.dot(t, wm_ref[...],
                             preferred_element_type=jnp.float32) + bm_ref[...]

    return pl.pallas_call(
        mk,
        grid=(nb,),
        in_specs=[
            pl.BlockSpec((27, rb, 128), lambda i: (0, i, 0)),
            pl.BlockSpec((27, 128, 256), lambda i: (0, 0, 0)),
            pl.BlockSpec((1, 256), lambda i: (0, 0)),
            pl.BlockSpec((256, 12), lambda i: (0, 0)),
            pl.BlockSpec((1, 12), lambda i: (0, 0)),
        ],
        out_specs=pl.BlockSpec((rb, 12), lambda i: (i, 0)),
        out_shape=jax.ShapeDtypeStruct((pad_n // 4, 12), jnp.float32),
    )(g3d, w3big, b3big, wmbig, bmbig)


def kernel(coords, feats, W1, b1, W2, b2, W3, b3, Wm, bm):
    n = coords.shape[0]
    pad_n = -(-n // (_NW * _L)) * (_NW * _L)
    pad = pad_n - n
    lin = coords[:, 0] * (_D * _D) + coords[:, 1] * _D + coords[:, 2]
    lin_a1 = jnp.concatenate(
        [lin, _SENT + 8 + jnp.arange(pad, dtype=jnp.int32)])
    lin_a2 = jnp.concatenate([lin, jnp.full((pad,), _SENT, jnp.int32)])
    vals = jnp.arange(pad_n, dtype=jnp.int32)

    lookup = _build_lookup(lin_a1, vals, pad_n)
    fidx = _neighbor_idx(lin_a2, lookup, pad_n)

    r = pad_n + _ZROWS
    x1 = jnp.pad(feats.reshape(-1), (0, r - n))
    g1 = _gather_scalars_local(x1, fidx, pad_n)
    x2 = _mm1(g1, W1, b1, pad_n)
    g2 = _gather_rows(x2, fidx, pad_n)
    x3 = _mm_mid(g2, W2, b2, pad_n, cin=16, cout=32, blk_p=512,
                 out_dtype=jnp.bfloat16)
    g3 = _gather_rows(x3, fidx, pad_n)
    out = _mm3_head(g3, W3, b3, Wm, bm, pad_n)
    return out.reshape(pad_n, 3)[:n]


# speedup vs baseline: 1.2681x; 1.2681x over previous
import functools

import jax
import jax.numpy as jnp
from jax import lax
from jax.experimental import pallas as pl
from jax.experimental.pallas import tpu as pltpu
from jax.experimental.pallas import tpu_sc as plsc

_D = 64
_BLK = 256
_NC, _NS, _L = 2, 16, 16
_NW = _NC * _NS
_SENT = _D * _D * _D
_T = 262656

_OFF = [(dx, dy, dz) for dx in (-1, 0, 1) for dy in (-1, 0, 1) for dz in (-1, 0, 1)]

_SC_PARAMS = pltpu.CompilerParams(use_tc_tiling_on_sc=False,
                                  needs_layout_passes=False)
_ZROWS = 16384


def _build_lookup(lin_a1, vals, pad_n):
    p1 = pad_n // _NS
    ts = _T // _NS
    mesh = plsc.VectorSubcoreMesh(
        core_axis_name="c", subcore_axis_name="s", num_cores=1)

    @functools.partial(
        pl.kernel,
        out_type=jax.ShapeDtypeStruct((_T,), jnp.int32),
        mesh=mesh,
        compiler_params=_SC_PARAMS,
        scratch_types=[
            pltpu.VMEM((ts,), jnp.int32),
            pltpu.VMEM((p1,), jnp.int32),
            pltpu.VMEM((p1,), jnp.int32),
            pltpu.SemaphoreType.DMA,
        ],
    )
    def k(lin_hbm, vals_hbm, lookup_hbm, fillv, linv, valsv, sem):
        wid = lax.axis_index("s")
        neg1 = jnp.full((_L,), -1, jnp.int32)

        def fill_body(i, c):
            fillv[pl.ds(i * _L, _L)] = neg1
            return c

        lax.fori_loop(0, ts // _L, fill_body, 0)
        pltpu.sync_copy(fillv, lookup_hbm.at[pl.ds(wid * ts, ts)])
        plsc.subcore_barrier()
        pltpu.sync_copy(lin_hbm.at[pl.ds(wid * p1, p1)], linv)
        pltpu.sync_copy(vals_hbm.at[pl.ds(wid * p1, p1)], valsv)
        pltpu.async_copy(valsv, lookup_hbm.at[linv], sem).wait()

    return k(lin_a1, vals)


def _neighbor_idx(lin_a2, lookup, pad_n):
    p = pad_n // _NW
    ng = p // _L
    e = p * 27
    zr = pad_n
    mesh = plsc.VectorSubcoreMesh(core_axis_name="c", subcore_axis_name="s")

    @functools.partial(
        pl.kernel,
        out_type=jax.ShapeDtypeStruct((27 * pad_n,), jnp.int32),
        mesh=mesh,
        compiler_params=_SC_PARAMS,
        scratch_types=[
            pltpu.VMEM((p,), jnp.int32),
            pltpu.VMEM((e,), jnp.int32),
            pltpu.VMEM((e,), jnp.int32),
            [pltpu.SemaphoreType.DMA] * 4,
        ],
    )
    def k(lin_hbm, lookup_hbm, fidx_hbm, linself, linbuf, rawbuf, sem):
        wid = lax.axis_index("s") * _NC + lax.axis_index("c")
        base = wid * p
        pltpu.sync_copy(lin_hbm.at[pl.ds(base, p)], linself)

        def g_body(g, c):
            lin16 = linself[pl.ds(g * _L, _L)]
            x = jnp.right_shift(lin16, 12)
            y = jnp.bitwise_and(jnp.right_shift(lin16, 6), 63)
            z = jnp.bitwise_and(lin16, 63)
            for kk, (dx, dy, dz) in enumerate(_OFF):
                inb = None
                for comp, dd in ((x, dx), (y, dy), (z, dz)):
                    if dd == -1:
                        m = comp >= 1
                    elif dd == 1:
                        m = comp <= _D - 2
                    else:
                        continue
                    inb = m if inb is None else jnp.logical_and(inb, m)
                nlin = lin16 + (dx * 4096 + dy * 64 + dz)
                if inb is not None:
                    nlin = jnp.where(inb, nlin, _SENT)
                linbuf[pl.ds(kk * p + g * _L, _L)] = nlin
            return c

        lax.fori_loop(0, ng, g_body, 0)
        quarter = e // 4
        descs = [
            pltpu.async_copy(
                lookup_hbm.at[linbuf.at[pl.ds(j * quarter, quarter)]],
                rawbuf.at[pl.ds(j * quarter, quarter)], sem[j])
            for j in range(4)
        ]
        for d in descs:
            d.wait()

        iota16 = lax.iota(jnp.int32, _L)

        def f_body(v, c):
            r = rawbuf[pl.ds(v * _L, _L)]
            spread = jnp.bitwise_and(v * _L + iota16, _ZROWS - 1)
            rawbuf[pl.ds(v * _L, _L)] = jnp.where(r >= 0, r, zr + spread)
            return c

        lax.fori_loop(0, e // _L, f_body, 0)
        for kk in range(27):
            pltpu.sync_copy(rawbuf.at[pl.ds(kk * p, p)],
                            fidx_hbm.at[pl.ds(kk * pad_n + base, p)])

    return k(lin_a2, lookup)


_RING = 4


def _gather_scalars_local(x_pad, fidx, pad_n):
    rows_n = fidx.shape[0]
    r = x_pad.shape[0]
    p = pad_n // _NW
    e = p * 27
    eq = e // 2
    mesh = plsc.VectorSubcoreMesh(core_axis_name="c", subcore_axis_name="s")

    @functools.partial(
        pl.kernel,
        out_type=jax.ShapeDtypeStruct((rows_n,), jnp.float32),
        mesh=mesh,
        compiler_params=_SC_PARAMS,
        scratch_types=[
            pltpu.VMEM((r,), jnp.float32),
            pltpu.VMEM((e,), jnp.int32),
            pltpu.VMEM((eq,), jnp.float32),
            pltpu.SemaphoreType.DMA,
        ],
    )
    def k(x_hbm, fidx_hbm, g_hbm, xloc, idxv, outv, sem):
        wid = lax.axis_index("s") * _NC + lax.axis_index("c")
        base = wid * e
        pltpu.sync_copy(x_hbm, xloc)
        pltpu.sync_copy(fidx_hbm.at[pl.ds(base, e)], idxv)

        def quarter(qi, c):
            def body(v, c2):
                i16 = idxv[pl.ds(qi * eq + v * _L, _L)]
                outv[pl.ds(v * _L, _L)] = plsc.load_gather(xloc, [i16])
                return c2

            lax.fori_loop(0, eq // _L, body, 0)
            pltpu.sync_copy(outv, g_hbm.at[pl.ds(base + qi * eq, eq)])
            return c

        lax.fori_loop(0, 2, quarter, 0)

    return k(x_pad, fidx)


def _gather_rows(x_pad, fidx, pad_n):
    rows_n = fidx.shape[0]
    flat = x_pad.ndim == 1
    c = 1 if flat else x_pad.shape[1]
    p = pad_n // _NW
    h = p // 2
    nch = 54
    dt = x_pad.dtype
    out_sds = jax.ShapeDtypeStruct((rows_n,) if flat else (rows_n, c), dt)
    mesh = plsc.VectorSubcoreMesh(core_axis_name="c", subcore_axis_name="s")

    @functools.partial(
        pl.kernel,
        out_type=out_sds,
        mesh=mesh,
        compiler_params=_SC_PARAMS,
        scratch_types=[
            [pltpu.VMEM((h,), jnp.int32)] * _RING,
            [pltpu.VMEM((h,) if flat else (h, c), dt)] * _RING,
            [pltpu.SemaphoreType.DMA] * _RING,
        ],
    )
    def k(x_hbm, fidx_hbm, g_hbm, idxs, rows, sems):
        wid = lax.axis_index("s") * _NC + lax.axis_index("c")
        base = wid * p

        def off(i):
            return (i // 2) * pad_n + base + (i % 2) * h

        descs = [None] * _RING

        def fire(i, b):
            pltpu.sync_copy(fidx_hbm.at[pl.ds(off(i), h)], idxs[b])
            descs[b] = pltpu.async_copy(x_hbm.at[idxs[b]], rows[b], sems[b])

        for r in range(_RING):
            fire(r, r)
        for i in range(nch):
            b = i % _RING
            descs[b].wait()
            pltpu.sync_copy(rows[b], g_hbm.at[pl.ds(off(i), h)])
            if i + _RING < nch:
                fire(i + _RING, b)

    return k(x_pad, fidx)


def _mm1(g1, w1, b1, pad_n):
    g3d = g1.reshape(27, pad_n // 128, 128)
    eye = jnp.eye(128, dtype=jnp.float32)
    r16 = jnp.repeat(eye, 16, axis=1)
    w1t = jnp.tile(w1[:, 0, :], (1, 128))
    b1t = jnp.tile(b1, 128).reshape(1, 2048)
    blk_p = 1024
    rb = blk_p // 128
    nb = pad_n // blk_p
    nz = _ZROWS // blk_p

    def mk(g_ref, r_ref, w_ref, b_ref, o_ref):
        pid = pl.program_id(0)

        @pl.when(pid < nb)
        def _():
            acc = jnp.zeros((rb, 2048), jnp.float32)
            for kk in range(27):
                e = jnp.dot(g_ref[kk], r_ref[...],
                            preferred_element_type=jnp.float32)
                acc += e * w_ref[kk]
            o_ref[...] = acc + b_ref[...]

        @pl.when(pid >= nb)
        def _():
            o_ref[...] = jnp.zeros_like(o_ref)

    out = pl.pallas_call(
        mk,
        grid=(nb + nz,),
        in_specs=[
            pl.BlockSpec((27, rb, 128), lambda i: (0, jnp.minimum(i, nb - 1), 0)),
            pl.BlockSpec((128, 2048), lambda i: (0, 0)),
            pl.BlockSpec((27, 2048), lambda i: (0, 0)),
            pl.BlockSpec((1, 2048), lambda i: (0, 0)),
        ],
        out_specs=pl.BlockSpec((rb, 2048), lambda i: (i, 0)),
        out_shape=jax.ShapeDtypeStruct(((pad_n + _ZROWS) // 128, 2048),
                                       jnp.float32),
    )(g3d, r16, w1t, b1t)
    return out.reshape(pad_n + _ZROWS, 16)


def _mm_mid(g, w, b, pad_n, cin, cout, blk_p, out_dtype=jnp.float32):
    ppr = 128 // cin
    opr = 256 // cout
    g3d = g.reshape(27, pad_n * cin // 128, 128)
    eye = jnp.eye(ppr, dtype=jnp.float32)
    wbig = jnp.einsum("jm,kcd->kjcmd", eye, w).reshape(27, 128, opr * cout)
    bbig = jnp.tile(b, opr).reshape(1, opr * cout)
    rb = blk_p * cin // 128
    orb = blk_p // opr
    nb = pad_n // blk_p
    nz = _ZROWS // blk_p

    def mk(g_ref, w_ref, b_ref, o_ref):
        pid = pl.program_id(0)

        @pl.when(pid < nb)
        def _():
            acc = jnp.zeros((rb, opr * cout), jnp.float32)
            for kk in range(27):
                acc += jnp.dot(g_ref[kk], w_ref[kk],
                               preferred_element_type=jnp.float32)
            o_ref[...] = (acc + b_ref[...]).astype(out_dtype)

        @pl.when(pid >= nb)
        def _():
            o_ref[...] = jnp.zeros_like(o_ref)

    out = pl.pallas_call(
        mk,
        grid=(nb + nz,),
        in_specs=[
            pl.BlockSpec((27, rb, 128), lambda i: (0, jnp.minimum(i, nb - 1), 0)),
            pl.BlockSpec((27, 128, opr * cout), lambda i: (0, 0, 0)),
            pl.BlockSpec((1, opr * cout), lambda i: (0, 0)),
        ],
        out_specs=pl.BlockSpec((orb, opr * cout), lambda i: (i, 0)),
        out_shape=jax.ShapeDtypeStruct(
            ((pad_n + _ZROWS) // opr, opr * cout), out_dtype),
    )(g3d, wbig, bbig)
    return out.reshape(pad_n + _ZROWS, cout)


def _mm3_head(g, w3, b3, wm, bm, pad_n):
    g3d = g.reshape(27, pad_n * 32 // 128, 128)
    eye4 = jnp.eye(4, dtype=jnp.float32)
    w3big = jnp.einsum("jm,kcd->kjcmd", eye4, w3).reshape(27, 128, 256)
    b3big = jnp.tile(b3, 4).reshape(1, 256)
    wmbig = jnp.einsum("jm,cd->jcmd", eye4, wm).reshape(256, 12)
    bmbig = jnp.tile(bm, 4).reshape(1, 12)
    blk_p = 256
    rb = blk_p * 32 // 128
    nb = pad_n // blk_p

    def mk(g_ref, w_ref, b_ref, wm_ref, bm_ref, o_ref):
        acc = jnp.zeros((rb, 256), jnp.float32)
        for kk in range(27):
            acc += jnp.dot(g_ref[kk].astype(jnp.float32), w_ref[kk],
                           preferred_element_type=jnp.float32)
        t = acc + b_ref[...]
        o_ref[...] = jnp.dot(t, wm_ref[...],
                             preferred_element_type=jnp.float32) + bm_ref[...]

    return pl.pallas_call(
        mk,
        grid=(nb,),
        in_specs=[
            pl.BlockSpec((27, rb, 128), lambda i: (0, i, 0)),
            pl.BlockSpec((27, 128, 256), lambda i: (0, 0, 0)),
            pl.BlockSpec((1, 256), lambda i: (0, 0)),
            pl.BlockSpec((256, 12), lambda i: (0, 0)),
            pl.BlockSpec((1, 12), lambda i: (0, 0)),
        ],
        out_specs=pl.BlockSpec((rb, 12), lambda i: (i, 0)),
        out_shape=jax.ShapeDtypeStruct((pad_n // 4, 12), jnp.float32),
    )(g3d, w3big, b3big, wmbig, bmbig)


def kernel(coords, feats, W1, b1, W2, b2, W3, b3, Wm, bm):
    n = coords.shape[0]
    pad_n = -(-n // (_NW * _L)) * (_NW * _L)
    pad = pad_n - n
    lin = coords[:, 0] * (_D * _D) + coords[:, 1] * _D + coords[:, 2]
    lin_a1 = jnp.concatenate(
        [lin, _SENT + 8 + jnp.arange(pad, dtype=jnp.int32)])
    lin_a2 = jnp.concatenate([lin, jnp.full((pad,), _SENT, jnp.int32)])
    vals = jnp.arange(pad_n, dtype=jnp.int32)

    lookup = _build_lookup(lin_a1, vals, pad_n)
    fidx = _neighbor_idx(lin_a2, lookup, pad_n)

    r = pad_n + _ZROWS
    x1 = jnp.pad(feats.reshape(-1), (0, r - n))
    g1 = _gather_scalars_local(x1, fidx, pad_n)
    x2 = _mm1(g1, W1, b1, pad_n)
    g2 = _gather_rows(x2, fidx, pad_n)
    x3 = _mm_mid(g2, W2, b2, pad_n, cin=16, cout=32, blk_p=512)
    g3 = _gather_rows(x3, fidx, pad_n)
    out = _mm3_head(g3, W3, b3, Wm, bm, pad_n)
    return out.reshape(pad_n, 3)[:n]


# speedup vs baseline: 1.5651x; 1.2342x over previous
import functools

import jax
import jax.numpy as jnp
from jax import lax
from jax.experimental import pallas as pl
from jax.experimental.pallas import tpu as pltpu
from jax.experimental.pallas import tpu_sc as plsc

_D = 64
_BLK = 256
_NC, _NS, _L = 2, 16, 16
_NW = _NC * _NS
_SENT = _D * _D * _D
_T = 262656

_OFF = [(dx, dy, dz) for dx in (-1, 0, 1) for dy in (-1, 0, 1) for dz in (-1, 0, 1)]

_SC_PARAMS = pltpu.CompilerParams(use_tc_tiling_on_sc=False,
                                  needs_layout_passes=False)
_ZROWS = 16384


def _build_lookup(lin_a1, vals, pad_n):
    p1 = pad_n // _NS
    ts = _T // _NS
    mesh = plsc.VectorSubcoreMesh(
        core_axis_name="c", subcore_axis_name="s", num_cores=1)

    @functools.partial(
        pl.kernel,
        out_type=jax.ShapeDtypeStruct((_T,), jnp.int32),
        mesh=mesh,
        compiler_params=_SC_PARAMS,
        scratch_types=[
            pltpu.VMEM((ts,), jnp.int32),
            pltpu.VMEM((p1,), jnp.int32),
            pltpu.VMEM((p1,), jnp.int32),
            pltpu.SemaphoreType.DMA,
        ],
    )
    def k(lin_hbm, vals_hbm, lookup_hbm, fillv, linv, valsv, sem):
        wid = lax.axis_index("s")
        neg1 = jnp.full((_L,), -1, jnp.int32)

        def fill_body(i, c):
            fillv[pl.ds(i * _L, _L)] = neg1
            return c

        lax.fori_loop(0, ts // _L, fill_body, 0)
        pltpu.sync_copy(fillv, lookup_hbm.at[pl.ds(wid * ts, ts)])
        plsc.subcore_barrier()
        pltpu.sync_copy(lin_hbm.at[pl.ds(wid * p1, p1)], linv)
        pltpu.sync_copy(vals_hbm.at[pl.ds(wid * p1, p1)], valsv)
        pltpu.async_copy(valsv, lookup_hbm.at[linv], sem).wait()

    return k(lin_a1, vals)


def _neighbor_idx(lin_a2, lookup, pad_n):
    p = pad_n // _NW
    ng = p // _L
    e = p * 27
    zr = pad_n
    mesh = plsc.VectorSubcoreMesh(core_axis_name="c", subcore_axis_name="s")

    @functools.partial(
        pl.kernel,
        out_type=jax.ShapeDtypeStruct((27 * pad_n,), jnp.int32),
        mesh=mesh,
        compiler_params=_SC_PARAMS,
        scratch_types=[
            pltpu.VMEM((p,), jnp.int32),
            pltpu.VMEM((e,), jnp.int32),
            pltpu.VMEM((e,), jnp.int32),
            pltpu.VMEM_SHARED((_T,), jnp.int32),
            [pltpu.SemaphoreType.DMA] * 4,
        ],
    )
    def k(lin_hbm, lookup_hbm, fidx_hbm, linself, linbuf, rawbuf, lookup_sp,
          sem):
        sid = lax.axis_index("s")
        wid = sid * _NC + lax.axis_index("c")
        base = wid * p
        ts = _T // _NS
        pltpu.sync_copy(lookup_hbm.at[pl.ds(sid * ts, ts)],
                        lookup_sp.at[pl.ds(sid * ts, ts)])
        pltpu.sync_copy(lin_hbm.at[pl.ds(base, p)], linself)

        def g_body(g, c):
            lin16 = linself[pl.ds(g * _L, _L)]
            x = jnp.right_shift(lin16, 12)
            y = jnp.bitwise_and(jnp.right_shift(lin16, 6), 63)
            z = jnp.bitwise_and(lin16, 63)
            for kk, (dx, dy, dz) in enumerate(_OFF):
                inb = None
                for comp, dd in ((x, dx), (y, dy), (z, dz)):
                    if dd == -1:
                        m = comp >= 1
                    elif dd == 1:
                        m = comp <= _D - 2
                    else:
                        continue
                    inb = m if inb is None else jnp.logical_and(inb, m)
                nlin = lin16 + (dx * 4096 + dy * 64 + dz)
                if inb is not None:
                    nlin = jnp.where(inb, nlin, _SENT)
                linbuf[pl.ds(kk * p + g * _L, _L)] = nlin
            return c

        lax.fori_loop(0, ng, g_body, 0)
        plsc.subcore_barrier()
        quarter = e // 4
        descs = [
            pltpu.async_copy(
                lookup_sp.at[linbuf.at[pl.ds(j * quarter, quarter)]],
                rawbuf.at[pl.ds(j * quarter, quarter)], sem[j])
            for j in range(4)
        ]
        for d in descs:
            d.wait()

        iota16 = lax.iota(jnp.int32, _L)

        def f_body(v, c):
            r = rawbuf[pl.ds(v * _L, _L)]
            spread = jnp.bitwise_and(v * _L + iota16, _ZROWS - 1)
            rawbuf[pl.ds(v * _L, _L)] = jnp.where(r >= 0, r, zr + spread)
            return c

        lax.fori_loop(0, e // _L, f_body, 0)
        for kk in range(27):
            pltpu.sync_copy(rawbuf.at[pl.ds(kk * p, p)],
                            fidx_hbm.at[pl.ds(kk * pad_n + base, p)])

    return k(lin_a2, lookup)


_RING = 4


def _gather_scalars_local(x_pad, fidx, pad_n):
    rows_n = fidx.shape[0]
    r = x_pad.shape[0]
    p = pad_n // _NW
    e = p * 27
    eq = e // 2
    mesh = plsc.VectorSubcoreMesh(core_axis_name="c", subcore_axis_name="s")

    @functools.partial(
        pl.kernel,
        out_type=jax.ShapeDtypeStruct((rows_n,), jnp.float32),
        mesh=mesh,
        compiler_params=_SC_PARAMS,
        scratch_types=[
            pltpu.VMEM((r,), jnp.float32),
            pltpu.VMEM((e,), jnp.int32),
            pltpu.VMEM((eq,), jnp.float32),
            pltpu.SemaphoreType.DMA,
        ],
    )
    def k(x_hbm, fidx_hbm, g_hbm, xloc, idxv, outv, sem):
        wid = lax.axis_index("s") * _NC + lax.axis_index("c")
        base = wid * e
        pltpu.sync_copy(x_hbm, xloc)
        pltpu.sync_copy(fidx_hbm.at[pl.ds(base, e)], idxv)

        def quarter(qi, c):
            def body(v, c2):
                i16 = idxv[pl.ds(qi * eq + v * _L, _L)]
                outv[pl.ds(v * _L, _L)] = plsc.load_gather(xloc, [i16])
                return c2

            lax.fori_loop(0, eq // _L, body, 0)
            pltpu.sync_copy(outv, g_hbm.at[pl.ds(base + qi * eq, eq)])
            return c

        lax.fori_loop(0, 2, quarter, 0)

    return k(x_pad, fidx)


def _gather_rows(x_pad, fidx, pad_n):
    rows_n = fidx.shape[0]
    flat = x_pad.ndim == 1
    c = 1 if flat else x_pad.shape[1]
    p = pad_n // _NW
    h = p // 2
    nch = 54
    dt = x_pad.dtype
    out_sds = jax.ShapeDtypeStruct((rows_n,) if flat else (rows_n, c), dt)
    mesh = plsc.VectorSubcoreMesh(core_axis_name="c", subcore_axis_name="s")

    @functools.partial(
        pl.kernel,
        out_type=out_sds,
        mesh=mesh,
        compiler_params=_SC_PARAMS,
        scratch_types=[
            [pltpu.VMEM((h,), jnp.int32)] * _RING,
            [pltpu.VMEM((h,) if flat else (h, c), dt)] * _RING,
            [pltpu.SemaphoreType.DMA] * _RING,
        ],
    )
    def k(x_hbm, fidx_hbm, g_hbm, idxs, rows, sems):
        wid = lax.axis_index("s") * _NC + lax.axis_index("c")
        base = wid * p

        def off(i):
            return (i // 2) * pad_n + base + (i % 2) * h

        descs = [None] * _RING

        def fire(i, b):
            pltpu.sync_copy(fidx_hbm.at[pl.ds(off(i), h)], idxs[b])
            descs[b] = pltpu.async_copy(x_hbm.at[idxs[b]], rows[b], sems[b])

        for r in range(_RING):
            fire(r, r)
        for i in range(nch):
            b = i % _RING
            descs[b].wait()
            pltpu.sync_copy(rows[b], g_hbm.at[pl.ds(off(i), h)])
            if i + _RING < nch:
                fire(i + _RING, b)

    return k(x_pad, fidx)


def _mm1(g1, w1, b1, pad_n):
    g3d = g1.reshape(27, pad_n // 128, 128)
    eye = jnp.eye(128, dtype=jnp.float32)
    r16 = jnp.repeat(eye, 16, axis=1)
    w1t = jnp.tile(w1[:, 0, :], (1, 128))
    b1t = jnp.tile(b1, 128).reshape(1, 2048)
    blk_p = 1024
    rb = blk_p // 128
    nb = pad_n // blk_p
    nz = _ZROWS // blk_p

    def mk(g_ref, r_ref, w_ref, b_ref, o_ref):
        pid = pl.program_id(0)

        @pl.when(pid < nb)
        def _():
            acc = jnp.zeros((rb, 2048), jnp.float32)
            for kk in range(27):
                e = jnp.dot(g_ref[kk], r_ref[...],
                            preferred_element_type=jnp.float32)
                acc += e * w_ref[kk]
            o_ref[...] = acc + b_ref[...]

        @pl.when(pid >= nb)
        def _():
            o_ref[...] = jnp.zeros_like(o_ref)

    out = pl.pallas_call(
        mk,
        grid=(nb + nz,),
        in_specs=[
            pl.BlockSpec((27, rb, 128), lambda i: (0, jnp.minimum(i, nb - 1), 0)),
            pl.BlockSpec((128, 2048), lambda i: (0, 0)),
            pl.BlockSpec((27, 2048), lambda i: (0, 0)),
            pl.BlockSpec((1, 2048), lambda i: (0, 0)),
        ],
        out_specs=pl.BlockSpec((rb, 2048), lambda i: (i, 0)),
        out_shape=jax.ShapeDtypeStruct(((pad_n + _ZROWS) // 128, 2048),
                                       jnp.float32),
    )(g3d, r16, w1t, b1t)
    return out.reshape(pad_n + _ZROWS, 16)


def _mm_mid(g, w, b, pad_n, cin, cout, blk_p, out_dtype=jnp.float32):
    ppr = 128 // cin
    opr = 256 // cout
    g3d = g.reshape(27, pad_n * cin // 128, 128)
    eye = jnp.eye(ppr, dtype=jnp.float32)
    wbig = jnp.einsum("jm,kcd->kjcmd", eye, w).reshape(27, 128, opr * cout)
    bbig = jnp.tile(b, opr).reshape(1, opr * cout)
    rb = blk_p * cin // 128
    orb = blk_p // opr
    nb = pad_n // blk_p
    nz = _ZROWS // blk_p

    def mk(g_ref, w_ref, b_ref, o_ref):
        pid = pl.program_id(0)

        @pl.when(pid < nb)
        def _():
            acc = jnp.zeros((rb, opr * cout), jnp.float32)
            for kk in range(27):
                acc += jnp.dot(g_ref[kk], w_ref[kk],
                               preferred_element_type=jnp.float32)
            o_ref[...] = (acc + b_ref[...]).astype(out_dtype)

        @pl.when(pid >= nb)
        def _():
            o_ref[...] = jnp.zeros_like(o_ref)

    out = pl.pallas_call(
        mk,
        grid=(nb + nz,),
        in_specs=[
            pl.BlockSpec((27, rb, 128), lambda i: (0, jnp.minimum(i, nb - 1), 0)),
            pl.BlockSpec((27, 128, opr * cout), lambda i: (0, 0, 0)),
            pl.BlockSpec((1, opr * cout), lambda i: (0, 0)),
        ],
        out_specs=pl.BlockSpec((orb, opr * cout), lambda i: (i, 0)),
        out_shape=jax.ShapeDtypeStruct(
            ((pad_n + _ZROWS) // opr, opr * cout), out_dtype),
    )(g3d, wbig, bbig)
    return out.reshape(pad_n + _ZROWS, cout)


def _mm3_head(g, w3, b3, wm, bm, pad_n):
    g3d = g.reshape(27, pad_n * 32 // 128, 128)
    eye4 = jnp.eye(4, dtype=jnp.float32)
    w3big = jnp.einsum("jm,kcd->kjcmd", eye4, w3).reshape(27, 128, 256)
    b3big = jnp.tile(b3, 4).reshape(1, 256)
    wmbig = jnp.einsum("jm,cd->jcmd", eye4, wm).reshape(256, 12)
    bmbig = jnp.tile(bm, 4).reshape(1, 12)
    blk_p = 256
    rb = blk_p * 32 // 128
    nb = pad_n // blk_p

    def mk(g_ref, w_ref, b_ref, wm_ref, bm_ref, o_ref):
        acc = jnp.zeros((rb, 256), jnp.float32)
        for kk in range(27):
            acc += jnp.dot(g_ref[kk].astype(jnp.float32), w_ref[kk],
                           preferred_element_type=jnp.float32)
        t = acc + b_ref[...]
        o_ref[...] = jnp.dot(t, wm_ref[...],
                             preferred_element_type=jnp.float32) + bm_ref[...]

    return pl.pallas_call(
        mk,
        grid=(nb,),
        in_specs=[
            pl.BlockSpec((27, rb, 128), lambda i: (0, i, 0)),
            pl.BlockSpec((27, 128, 256), lambda i: (0, 0, 0)),
            pl.BlockSpec((1, 256), lambda i: (0, 0)),
            pl.BlockSpec((256, 12), lambda i: (0, 0)),
            pl.BlockSpec((1, 12), lambda i: (0, 0)),
        ],
        out_specs=pl.BlockSpec((rb, 12), lambda i: (i, 0)),
        out_shape=jax.ShapeDtypeStruct((pad_n // 4, 12), jnp.float32),
    )(g3d, w3big, b3big, wmbig, bmbig)


def kernel(coords, feats, W1, b1, W2, b2, W3, b3, Wm, bm):
    n = coords.shape[0]
    pad_n = -(-n // (_NW * _L)) * (_NW * _L)
    pad = pad_n - n
    lin = coords[:, 0] * (_D * _D) + coords[:, 1] * _D + coords[:, 2]
    lin_a1 = jnp.concatenate(
        [lin, _SENT + 8 + jnp.arange(pad, dtype=jnp.int32)])
    lin_a2 = jnp.concatenate([lin, jnp.full((pad,), _SENT, jnp.int32)])
    vals = jnp.arange(pad_n, dtype=jnp.int32)

    lookup = _build_lookup(lin_a1, vals, pad_n)
    fidx = _neighbor_idx(lin_a2, lookup, pad_n)

    r = pad_n + _ZROWS
    x1 = jnp.pad(feats.reshape(-1), (0, r - n))
    g1 = _gather_scalars_local(x1, fidx, pad_n)
    x2 = _mm1(g1, W1, b1, pad_n)
    g2 = _gather_rows(x2, fidx, pad_n)
    x3 = _mm_mid(g2, W2, b2, pad_n, cin=16, cout=32, blk_p=512)
    g3 = _gather_rows(x3, fidx, pad_n)
    out = _mm3_head(g3, W3, b3, Wm, bm, pad_n)
    return out.reshape(pad_n, 3)[:n]


# speedup vs baseline: 1.6929x; 1.0817x over previous
import functools

import jax
import jax.numpy as jnp
from jax import lax
from jax.experimental import pallas as pl
from jax.experimental.pallas import tpu as pltpu
from jax.experimental.pallas import tpu_sc as plsc

_D = 64
_BLK = 256
_NC, _NS, _L = 2, 16, 16
_NW = _NC * _NS
_SENT = _D * _D * _D
_T = 262656

_OFF = [(dx, dy, dz) for dx in (-1, 0, 1) for dy in (-1, 0, 1) for dz in (-1, 0, 1)]

_SC_PARAMS = pltpu.CompilerParams(use_tc_tiling_on_sc=False,
                                  needs_layout_passes=False)
_ZROWS = 16384


def _neighbor_idx(lin_a1, lin_a2, vals, pad_n):
    p = pad_n // _NW
    ng = p // _L
    e = p * 27
    zr = pad_n
    p2 = pad_n // _NS
    ts = _T // _NS
    mesh = plsc.VectorSubcoreMesh(core_axis_name="c", subcore_axis_name="s")

    @functools.partial(
        pl.kernel,
        out_type=jax.ShapeDtypeStruct((27 * pad_n,), jnp.int32),
        mesh=mesh,
        compiler_params=_SC_PARAMS,
        scratch_types=[
            pltpu.VMEM((p,), jnp.int32),
            pltpu.VMEM((e,), jnp.int32),
            pltpu.VMEM((e,), jnp.int32),
            pltpu.VMEM((ts,), jnp.int32),
            pltpu.VMEM((p2,), jnp.int32),
            pltpu.VMEM((p2,), jnp.int32),
            pltpu.VMEM_SHARED((_T,), jnp.int32),
            [pltpu.SemaphoreType.DMA] * 4,
            pltpu.SemaphoreType.DMA,
        ],
    )
    def k(lin1_hbm, lin2_hbm, vals_hbm, fidx_hbm, linself, linbuf, rawbuf,
          fillv, lin1v, valsv, lookup_sp, sem, sem_s):
        sid = lax.axis_index("s")
        wid = sid * _NC + lax.axis_index("c")
        base = wid * p
        neg1 = jnp.full((_L,), -1, jnp.int32)

        def fill_body(i, c):
            fillv[pl.ds(i * _L, _L)] = neg1
            return c

        lax.fori_loop(0, ts // _L, fill_body, 0)
        pltpu.sync_copy(fillv, lookup_sp.at[pl.ds(sid * ts, ts)])
        pltpu.sync_copy(lin1_hbm.at[pl.ds(sid * p2, p2)], lin1v)
        pltpu.sync_copy(vals_hbm.at[pl.ds(sid * p2, p2)], valsv)
        pltpu.sync_copy(lin2_hbm.at[pl.ds(base, p)], linself)
        plsc.subcore_barrier()
        scat = pltpu.async_copy(valsv, lookup_sp.at[lin1v], sem_s)

        def g_body(g, c):
            lin16 = linself[pl.ds(g * _L, _L)]
            x = jnp.right_shift(lin16, 12)
            y = jnp.bitwise_and(jnp.right_shift(lin16, 6), 63)
            z = jnp.bitwise_and(lin16, 63)
            for kk, (dx, dy, dz) in enumerate(_OFF):
                inb = None
                for comp, dd in ((x, dx), (y, dy), (z, dz)):
                    if dd == -1:
                        m = comp >= 1
                    elif dd == 1:
                        m = comp <= _D - 2
                    else:
                        continue
                    inb = m if inb is None else jnp.logical_and(inb, m)
                nlin = lin16 + (dx * 4096 + dy * 64 + dz)
                if inb is not None:
                    nlin = jnp.where(inb, nlin, _SENT)
                linbuf[pl.ds(kk * p + g * _L, _L)] = nlin
            return c

        lax.fori_loop(0, ng, g_body, 0)
        scat.wait()
        plsc.subcore_barrier()
        quarter = e // 4
        descs = [
            pltpu.async_copy(
                lookup_sp.at[linbuf.at[pl.ds(j * quarter, quarter)]],
                rawbuf.at[pl.ds(j * quarter, quarter)], sem[j])
            for j in range(4)
        ]
        for d in descs:
            d.wait()

        iota16 = lax.iota(jnp.int32, _L)

        def f_body(v, c):
            r = rawbuf[pl.ds(v * _L, _L)]
            spread = jnp.bitwise_and(v * _L + iota16, _ZROWS - 1)
            rawbuf[pl.ds(v * _L, _L)] = jnp.where(r >= 0, r, zr + spread)
            return c

        lax.fori_loop(0, e // _L, f_body, 0)
        for kk in range(27):
            pltpu.sync_copy(rawbuf.at[pl.ds(kk * p, p)],
                            fidx_hbm.at[pl.ds(kk * pad_n + base, p)])

    return k(lin_a1, lin_a2, vals)


_RING = 4


def _gather_scalars_local(x_pad, fidx, pad_n):
    rows_n = fidx.shape[0]
    r = x_pad.shape[0]
    p = pad_n // _NW
    e = p * 27
    eq = e // 2
    mesh = plsc.VectorSubcoreMesh(core_axis_name="c", subcore_axis_name="s")

    @functools.partial(
        pl.kernel,
        out_type=jax.ShapeDtypeStruct((rows_n,), jnp.float32),
        mesh=mesh,
        compiler_params=_SC_PARAMS,
        scratch_types=[
            pltpu.VMEM((r,), jnp.float32),
            pltpu.VMEM((e,), jnp.int32),
            pltpu.VMEM((eq,), jnp.float32),
            pltpu.SemaphoreType.DMA,
        ],
    )
    def k(x_hbm, fidx_hbm, g_hbm, xloc, idxv, outv, sem):
        wid = lax.axis_index("s") * _NC + lax.axis_index("c")
        base = wid * e
        pltpu.sync_copy(x_hbm, xloc)
        pltpu.sync_copy(fidx_hbm.at[pl.ds(base, e)], idxv)

        def quarter(qi, c):
            def body(v, c2):
                i16 = idxv[pl.ds(qi * eq + v * _L, _L)]
                outv[pl.ds(v * _L, _L)] = plsc.load_gather(xloc, [i16])
                return c2

            lax.fori_loop(0, eq // _L, body, 0)
            pltpu.sync_copy(outv, g_hbm.at[pl.ds(base + qi * eq, eq)])
            return c

        lax.fori_loop(0, 2, quarter, 0)

    return k(x_pad, fidx)


def _gather_rows(x_pad, fidx, pad_n):
    rows_n = fidx.shape[0]
    flat = x_pad.ndim == 1
    c = 1 if flat else x_pad.shape[1]
    p = pad_n // _NW
    h = p // 2
    nch = 54
    dt = x_pad.dtype
    out_sds = jax.ShapeDtypeStruct((rows_n,) if flat else (rows_n, c), dt)
    mesh = plsc.VectorSubcoreMesh(core_axis_name="c", subcore_axis_name="s")

    @functools.partial(
        pl.kernel,
        out_type=out_sds,
        mesh=mesh,
        compiler_params=_SC_PARAMS,
        scratch_types=[
            [pltpu.VMEM((h,), jnp.int32)] * _RING,
            [pltpu.VMEM((h,) if flat else (h, c), dt)] * _RING,
            [pltpu.SemaphoreType.DMA] * _RING,
        ],
    )
    def k(x_hbm, fidx_hbm, g_hbm, idxs, rows, sems):
        wid = lax.axis_index("s") * _NC + lax.axis_index("c")
        base = wid * p

        def off(i):
            return (i // 2) * pad_n + base + (i % 2) * h

        descs = [None] * _RING

        def fire(i, b):
            pltpu.sync_copy(fidx_hbm.at[pl.ds(off(i), h)], idxs[b])
            descs[b] = pltpu.async_copy(x_hbm.at[idxs[b]], rows[b], sems[b])

        for r in range(_RING):
            fire(r, r)
        for i in range(nch):
            b = i % _RING
            descs[b].wait()
            pltpu.sync_copy(rows[b], g_hbm.at[pl.ds(off(i), h)])
            if i + _RING < nch:
                fire(i + _RING, b)

    return k(x_pad, fidx)


def _mm1(g1, w1, b1, pad_n):
    g3d = g1.reshape(27, pad_n // 128, 128)
    eye = jnp.eye(128, dtype=jnp.float32)
    r16 = jnp.repeat(eye, 16, axis=1)
    w1t = jnp.tile(w1[:, 0, :], (1, 128))
    b1t = jnp.tile(b1, 128).reshape(1, 2048)
    blk_p = 1024
    rb = blk_p // 128
    nb = pad_n // blk_p
    nz = _ZROWS // blk_p

    def mk(g_ref, r_ref, w_ref, b_ref, o_ref):
        pid = pl.program_id(0)

        @pl.when(pid < nb)
        def _():
            acc = jnp.zeros((rb, 2048), jnp.float32)
            for kk in range(27):
                e = jnp.dot(g_ref[kk], r_ref[...],
                            preferred_element_type=jnp.float32)
                acc += e * w_ref[kk]
            o_ref[...] = acc + b_ref[...]

        @pl.when(pid >= nb)
        def _():
            o_ref[...] = jnp.zeros_like(o_ref)

    out = pl.pallas_call(
        mk,
        grid=(nb + nz,),
        in_specs=[
            pl.BlockSpec((27, rb, 128), lambda i: (0, jnp.minimum(i, nb - 1), 0)),
            pl.BlockSpec((128, 2048), lambda i: (0, 0)),
            pl.BlockSpec((27, 2048), lambda i: (0, 0)),
            pl.BlockSpec((1, 2048), lambda i: (0, 0)),
        ],
        out_specs=pl.BlockSpec((rb, 2048), lambda i: (i, 0)),
        out_shape=jax.ShapeDtypeStruct(((pad_n + _ZROWS) // 128, 2048),
                                       jnp.float32),
    )(g3d, r16, w1t, b1t)
    return out.reshape(pad_n + _ZROWS, 16)


def _mm_mid(g, w, b, pad_n, cin, cout, blk_p, out_dtype=jnp.float32):
    ppr = 128 // cin
    opr = 256 // cout
    g3d = g.reshape(27, pad_n * cin // 128, 128)
    eye = jnp.eye(ppr, dtype=jnp.float32)
    wbig = jnp.einsum("jm,kcd->kjcmd", eye, w).reshape(27, 128, opr * cout)
    bbig = jnp.tile(b, opr).reshape(1, opr * cout)
    rb = blk_p * cin // 128
    orb = blk_p // opr
    nb = pad_n // blk_p
    nz = _ZROWS // blk_p

    def mk(g_ref, w_ref, b_ref, o_ref):
        pid = pl.program_id(0)

        @pl.when(pid < nb)
        def _():
            acc = jnp.zeros((rb, opr * cout), jnp.float32)
            for kk in range(27):
                acc += jnp.dot(g_ref[kk], w_ref[kk],
                               preferred_element_type=jnp.float32)
            o_ref[...] = (acc + b_ref[...]).astype(out_dtype)

        @pl.when(pid >= nb)
        def _():
            o_ref[...] = jnp.zeros_like(o_ref)

    out = pl.pallas_call(
        mk,
        grid=(nb + nz,),
        in_specs=[
            pl.BlockSpec((27, rb, 128), lambda i: (0, jnp.minimum(i, nb - 1), 0)),
            pl.BlockSpec((27, 128, opr * cout), lambda i: (0, 0, 0)),
            pl.BlockSpec((1, opr * cout), lambda i: (0, 0)),
        ],
        out_specs=pl.BlockSpec((orb, opr * cout), lambda i: (i, 0)),
        out_shape=jax.ShapeDtypeStruct(
            ((pad_n + _ZROWS) // opr, opr * cout), out_dtype),
    )(g3d, wbig, bbig)
    return out.reshape(pad_n + _ZROWS, cout)


def _mm3_head(g, w3, b3, wm, bm, pad_n):
    g3d = g.reshape(27, pad_n * 32 // 128, 128)
    eye4 = jnp.eye(4, dtype=jnp.float32)
    w3big = jnp.einsum("jm,kcd->kjcmd", eye4, w3).reshape(27, 128, 256)
    b3big = jnp.tile(b3, 4).reshape(1, 256)
    wmbig = jnp.einsum("jm,cd->jcmd", eye4, wm).reshape(256, 12)
    bmbig = jnp.tile(bm, 4).reshape(1, 12)
    blk_p = 256
    rb = blk_p * 32 // 128
    nb = pad_n // blk_p

    def mk(g_ref, w_ref, b_ref, wm_ref, bm_ref, o_ref):
        acc = jnp.zeros((rb, 256), jnp.float32)
        for kk in range(27):
            acc += jnp.dot(g_ref[kk].astype(jnp.float32), w_ref[kk],
                           preferred_element_type=jnp.float32)
        t = acc + b_ref[...]
        o_ref[...] = jnp.dot(t, wm_ref[...],
                             preferred_element_type=jnp.float32) + bm_ref[...]

    return pl.pallas_call(
        mk,
        grid=(nb,),
        in_specs=[
            pl.BlockSpec((27, rb, 128), lambda i: (0, i, 0)),
            pl.BlockSpec((27, 128, 256), lambda i: (0, 0, 0)),
            pl.BlockSpec((1, 256), lambda i: (0, 0)),
            pl.BlockSpec((256, 12), lambda i: (0, 0)),
            pl.BlockSpec((1, 12), lambda i: (0, 0)),
        ],
        out_specs=pl.BlockSpec((rb, 12), lambda i: (i, 0)),
        out_shape=jax.ShapeDtypeStruct((pad_n // 4, 12), jnp.float32),
    )(g3d, w3big, b3big, wmbig, bmbig)


def kernel(coords, feats, W1, b1, W2, b2, W3, b3, Wm, bm):
    n = coords.shape[0]
    pad_n = -(-n // (_NW * _L)) * (_NW * _L)
    pad = pad_n - n
    lin = coords[:, 0] * (_D * _D) + coords[:, 1] * _D + coords[:, 2]
    lin_a1 = jnp.concatenate(
        [lin, _SENT + 8 + jnp.arange(pad, dtype=jnp.int32)])
    lin_a2 = jnp.concatenate([lin, jnp.full((pad,), _SENT, jnp.int32)])
    vals = jnp.arange(pad_n, dtype=jnp.int32)

    fidx = _neighbor_idx(lin_a1, lin_a2, vals, pad_n)

    r = pad_n + _ZROWS
    x1 = jnp.pad(feats.reshape(-1), (0, r - n))
    g1 = _gather_scalars_local(x1, fidx, pad_n)
    x2 = _mm1(g1, W1, b1, pad_n)
    g2 = _gather_rows(x2, fidx, pad_n)
    x3 = _mm_mid(g2, W2, b2, pad_n, cin=16, cout=32, blk_p=512)
    g3 = _gather_rows(x3, fidx, pad_n)
    out = _mm3_head(g3, W3, b3, Wm, bm, pad_n)
    return out.reshape(pad_n, 3)[:n]
